# baseline (speedup 1.0000x reference)
import functools

import jax
import jax.numpy as jnp
from jax import lax
from jax.experimental import pallas as pl
from jax.experimental.pallas import tpu as pltpu
from jax.experimental.pallas import tpu_sc as plsc

B = 16384
D = 64
V2 = 5000
NC = 2
NS = 16
NW = NC * NS
BPW = B // NW
CH = 128
NCH = BPW // CH
L = 16

_mesh = plsc.VectorSubcoreMesh(core_axis_name="c", subcore_axis_name="s")


@functools.partial(
    pl.kernel,
    mesh=_mesh,
    out_type=jax.ShapeDtypeStruct((B, D), jnp.float32),
    scratch_types=[
        pltpu.VMEM((BPW,), jnp.int32),
        pltpu.VMEM((BPW,), jnp.int32),
        pltpu.VMEM((BPW, 2 * D), jnp.float32),
        pltpu.VMEM((2 * CH, D), jnp.float32),
        pltpu.SemaphoreType.DMA((NCH,)),
        pltpu.SemaphoreType.DMA((2,)),
    ],
    compiler_params=pltpu.CompilerParams(needs_layout_passes=False),
)
def _gather_pairs(idx_hbm, table2_hbm, out_hbm, idx_v, idx2_v, pairs_v,
                  out_v, gsem, ssem):
    wid = lax.axis_index("s") * NC + lax.axis_index("c")
    base = wid * BPW
    pltpu.sync_copy(idx_hbm.at[pl.ds(base, BPW)], idx_v)
    for i in range(BPW // L):
        idx2_v[pl.ds(i * L, L)] = lax.shift_right_logical(
            idx_v[pl.ds(i * L, L)], 1)
    gathers = []
    for j in range(NCH):
        gathers.append(
            pltpu.async_copy(
                table2_hbm.at[idx2_v.at[pl.ds(j * CH, CH)]],
                pairs_v.at[pl.ds(j * CH, CH)],
                gsem.at[j],
            )
        )
    lanes = lax.iota(jnp.int32, L)
    stores = [None, None]
    for j in range(NCH):
        b = j % 2
        if stores[b] is not None:
            stores[b].wait()
        gathers[j].wait()

        def _group(g, _):
            i0 = j * CH + g * L
            rvec = lanes + i0
            ovec = lanes + (b * CH + g * L)
            half = (idx_v[pl.ds(i0, L)] & 1) * D
            for c in range(D):
                vals = plsc.load_gather(pairs_v, [rvec, half + c])
                plsc.store_scatter(
                    out_v, [ovec, jnp.full((L,), c, jnp.int32)], vals)
            return _

        lax.fori_loop(0, CH // L, _group, None)
        stores[b] = pltpu.async_copy(
            out_v.at[pl.ds(b * CH, CH)],
            out_hbm.at[pl.ds(base + j * CH, CH)],
            ssem.at[b],
        )
    for s in stores:
        if s is not None:
            s.wait()


def kernel(inputs, bias):
    idx = inputs.reshape(B).astype(jnp.int32)
    table2 = bias.reshape(V2, 2 * D)
    return _gather_pairs(idx, table2)


# speedup vs baseline: 2.6413x; 2.6413x over previous
import functools

import jax
import jax.numpy as jnp
from jax import lax
from jax.experimental import pallas as pl
from jax.experimental.pallas import tpu as pltpu
from jax.experimental.pallas import tpu_sc as plsc

B = 16384
D = 64
V = 10000
NC = 2
NS = 16
FB = 8
NFB = D // FB
NQ = 4
QB = B // NQ
L = 16

_mesh = plsc.VectorSubcoreMesh(core_axis_name="c", subcore_axis_name="s")


@functools.partial(
    pl.kernel,
    mesh=_mesh,
    out_type=jax.ShapeDtypeStruct((D, B), jnp.float32),
    scratch_types=[
        pltpu.VMEM((QB,), jnp.int32),
        pltpu.VMEM((FB, V), jnp.float32),
        pltpu.VMEM((FB, QB), jnp.float32),
        pltpu.SemaphoreType.DMA,
        pltpu.SemaphoreType.DMA,
    ],
    compiler_params=pltpu.CompilerParams(needs_layout_passes=False),
)
def _gather_fm(idxT_hbm, tableT_hbm, outT_hbm, idx_v, tab_v, out_v,
               isem, tsem):
    wid = lax.axis_index("s") * NC + lax.axis_index("c")
    fb = wid % NFB
    q = wid // NFB
    f0 = fb * FB
    b0 = q * QB
    c_idx = pltpu.async_copy(idxT_hbm.at[0, pl.ds(b0, QB)], idx_v, isem)
    c_tab = pltpu.async_copy(tableT_hbm.at[pl.ds(f0, FB), :], tab_v, tsem)
    c_idx.wait()
    c_tab.wait()

    @plsc.parallel_loop(0, QB, step=L)
    def _body(i):
        iv = idx_v[pl.ds(i, L)]
        for f in range(FB):
            vals = plsc.load_gather(
                tab_v, [jnp.full((L,), f, jnp.int32), iv])
            out_v[f, pl.ds(i, L)] = vals

    pltpu.sync_copy(out_v, outT_hbm.at[pl.ds(f0, FB), pl.ds(b0, QB)])


def kernel(inputs, bias):
    outT = _gather_fm(inputs.T.astype(jnp.int32), bias.T)
    return outT.T


# speedup vs baseline: 2.8375x; 1.0743x over previous
import functools

import jax
import jax.numpy as jnp
from jax import lax
from jax.experimental import pallas as pl
from jax.experimental.pallas import tpu as pltpu
from jax.experimental.pallas import tpu_sc as plsc

B = 16384
D = 64
V = 10000
NC = 2
NS = 16
FW = 4
NFW = D // FW
NH = 2
HB = B // NH
CHB = 2048
NCHB = HB // CHB
L = 16

_mesh = plsc.VectorSubcoreMesh(core_axis_name="c", subcore_axis_name="s")


@functools.partial(
    pl.kernel,
    mesh=_mesh,
    out_type=jax.ShapeDtypeStruct((D, B), jnp.float32),
    scratch_types=[
        pltpu.VMEM((HB,), jnp.int32),
        pltpu.VMEM((FW, V), jnp.float32),
        pltpu.VMEM((2, FW, CHB), jnp.float32),
        pltpu.SemaphoreType.DMA,
        pltpu.SemaphoreType.DMA,
        pltpu.SemaphoreType.DMA((2,)),
    ],
    compiler_params=pltpu.CompilerParams(needs_layout_passes=False),
)
def _gather_fm(idxT_hbm, tableT_hbm, outT_hbm, idx_v, tab_v, out_v,
               isem, tsem, ssem):
    wid = lax.axis_index("s") * NC + lax.axis_index("c")
    f0 = (wid % NFW) * FW
    b0 = (wid // NFW) * HB
    c_idx = pltpu.async_copy(idxT_hbm.at[0, pl.ds(b0, HB)], idx_v, isem)
    c_tab = pltpu.async_copy(tableT_hbm.at[pl.ds(f0, FW), :], tab_v, tsem)
    c_idx.wait()
    c_tab.wait()
    stores = [None, None]
    for ch in range(NCHB):
        buf = ch % 2
        if stores[buf] is not None:
            stores[buf].wait()

        @plsc.parallel_loop(0, CHB, step=L)
        def _body(i, _ch=ch, _buf=buf):
            iv = idx_v[pl.ds(_ch * CHB + i, L)]
            for f in range(FW):
                vals = plsc.load_gather(
                    tab_v, [jnp.full((L,), f, jnp.int32), iv])
                out_v[_buf, f, pl.ds(i, L)] = vals

        stores[buf] = pltpu.async_copy(
            out_v.at[buf],
            outT_hbm.at[pl.ds(f0, FW), pl.ds(b0 + ch * CHB, CHB)],
            ssem.at[buf],
        )
    for s in stores:
        if s is not None:
            s.wait()


def kernel(inputs, bias):
    outT = _gather_fm(inputs.T.astype(jnp.int32), bias.T)
    return outT.T
